# Initial kernel scaffold; baseline (speedup 1.0000x reference)
import jax
import jax.numpy as jnp
from jax.experimental import pallas as pl


def kernel(x, edge_index, edge_attr, fc1_W, fc1_b, k_W1, k_b1, k_W2, k_b2, root, conv_b, fc2_W, fc2_b):
    raise NotImplementedError("write your pallas kernel here")


# speedup vs baseline: 1.4875x; 1.4875x over previous
import functools

import jax
import jax.numpy as jnp
from jax import lax
from jax.experimental import pallas as pl
from jax.experimental.pallas import tpu as pltpu
from jax.experimental.pallas import tpu_sc as plsc

F32 = jnp.float32

NC = 2
NS = 16
NW = NC * NS
CH = 128
SB = 1024

WIDTH = 32
KR = 16
TE = 2048
TN = 2048


def _sc_mesh():
    return plsc.VectorSubcoreMesh(core_axis_name="c", subcore_axis_name="s",
                                  num_cores=NC, num_subcores=NS)


def _make_gather(n_pad, e_pad):
    nsb = e_pad // (NW * SB)
    ncpw = e_pad // (NW * CH)
    rps = n_pad // NS

    @functools.partial(
        pl.kernel,
        out_type=jax.ShapeDtypeStruct((e_pad, WIDTH), F32),
        mesh=_sc_mesh(),
        compiler_params=pltpu.CompilerParams(use_tc_tiling_on_sc=False),
        scratch_types=[
            pltpu.VMEM((ncpw, CH), jnp.int32),
            pltpu.VMEM((SB, WIDTH), F32),
            pltpu.VMEM_SHARED((n_pad, WIDTH), F32),
            pltpu.SemaphoreType.DMA,
        ],
    )
    def gather(table_hbm, idx_hbm, out_hbm, idx_v, rows_v, tbl, sem):
        c = lax.axis_index("c")
        s = lax.axis_index("s")
        wid = s * NC + c
        pltpu.sync_copy(table_hbm.at[pl.ds(s * rps, rps)],
                        tbl.at[pl.ds(s * rps, rps)])
        pltpu.sync_copy(idx_hbm.at[wid], idx_v)
        plsc.subcore_barrier()

        def sb_body(sb, carry):
            cps = [
                pltpu.async_copy(
                    tbl.at[idx_v.at[sb * (SB // CH) + j]],
                    rows_v.at[pl.ds(j * CH, CH)],
                    sem,
                )
                for j in range(SB // CH)
            ]
            for cp in cps:
                cp.wait()
            pltpu.sync_copy(
                rows_v, out_hbm.at[pl.ds((wid * nsb + sb) * SB, SB)]
            )
            return carry

        lax.fori_loop(0, nsb, sb_body, 0)

    return gather


def _make_scatter(n_pad, e_pad, width):
    nsb = e_pad // (NW * SB)
    ncpw = e_pad // (NW * CH)
    rps = n_pad // NS

    @functools.partial(
        pl.kernel,
        out_type=jax.ShapeDtypeStruct((NC, n_pad, width), F32),
        mesh=_sc_mesh(),
        compiler_params=pltpu.CompilerParams(use_tc_tiling_on_sc=False),
        scratch_types=[
            pltpu.VMEM((ncpw, CH), jnp.int32),
            pltpu.VMEM((SB, width), F32),
            pltpu.VMEM_SHARED((n_pad, width), F32),
        ],
    )
    def scatter(rows_hbm, idx_hbm, zeros_hbm, out_hbm, idx_v, mbuf, acc):
        c = lax.axis_index("c")
        s = lax.axis_index("s")
        wid = s * NC + c
        pltpu.sync_copy(idx_hbm.at[wid], idx_v)
        for i in range(rps // CH):
            pltpu.sync_copy(zeros_hbm, acc.at[pl.ds(s * rps + i * CH, CH)])
        plsc.subcore_barrier()

        def sb_body(sb, carry):
            pltpu.sync_copy(
                rows_hbm.at[pl.ds((wid * nsb + sb) * SB, SB)], mbuf
            )
            for j in range(SB // CH):
                pltpu.sync_copy(
                    mbuf.at[pl.ds(j * CH, CH)],
                    acc.at[idx_v.at[sb * (SB // CH) + j]],
                    add=True,
                )
            return carry

        lax.fori_loop(0, nsb, sb_body, 0)
        plsc.subcore_barrier()
        pltpu.sync_copy(
            acc.at[pl.ds(s * rps, rps)],
            out_hbm.at[c].at[pl.ds(s * rps, rps)],
        )

    return scatter


def _make_count(n_pad, e_pad):
    width = 16
    ncpw = e_pad // (NW * CH)
    rps = n_pad // NS

    @functools.partial(
        pl.kernel,
        out_type=jax.ShapeDtypeStruct((NC, n_pad, width), F32),
        mesh=_sc_mesh(),
        compiler_params=pltpu.CompilerParams(use_tc_tiling_on_sc=False),
        scratch_types=[
            pltpu.VMEM((ncpw, CH), jnp.int32),
            pltpu.VMEM((CH, width), F32),
            pltpu.VMEM_SHARED((n_pad, width), F32),
        ],
    )
    def count(idx_hbm, zeros_hbm, ones_hbm, out_hbm, idx_v, onesb, acc):
        c = lax.axis_index("c")
        s = lax.axis_index("s")
        pltpu.sync_copy(idx_hbm.at[s * NC + c], idx_v)
        pltpu.sync_copy(ones_hbm, onesb)
        for i in range(rps // CH):
            pltpu.sync_copy(zeros_hbm, acc.at[pl.ds(s * rps + i * CH, CH)])
        plsc.subcore_barrier()

        def ch_body(k, carry):
            pltpu.sync_copy(onesb, acc.at[idx_v.at[k]], add=True)
            return carry

        lax.fori_loop(0, ncpw, ch_body, 0)
        plsc.subcore_barrier()
        pltpu.sync_copy(
            acc.at[pl.ds(s * rps, rps)],
            out_hbm.at[c].at[pl.ds(s * rps, rps)],
        )

    return count


def _msg_body(hs_ref, kh_ref, a_ref, out_ref):
    hs = hs_ref[...]
    kh = kh_ref[...]
    acc = jnp.zeros((TE, WIDTH), F32)
    for r in range(9):
        pr = lax.dot_general(
            hs, a_ref[pl.ds(r * WIDTH, WIDTH), :],
            (((1,), (0,)), ((), ())), preferred_element_type=F32,
            precision=lax.Precision.HIGHEST,
        )
        acc = acc + pr * kh[:, r:r + 1]
    out_ref[...] = acc


def _msg_call(e_pad, hs, kh_aug, a_stack):
    grid = e_pad // TE
    return pl.pallas_call(
        _msg_body,
        grid=(grid,),
        in_specs=[
            pl.BlockSpec((TE, WIDTH), lambda i: (i, 0)),
            pl.BlockSpec((TE, KR), lambda i: (i, 0)),
            pl.BlockSpec((9 * WIDTH, WIDTH), lambda i: (0, 0)),
        ],
        out_specs=pl.BlockSpec((TE, WIDTH), lambda i: (i, 0)),
        out_shape=jax.ShapeDtypeStruct((e_pad, WIDTH), F32),
    )(hs, kh_aug, a_stack)


def _update_body(last, h_ref, agg_ref, cnt_ref, root_ref, cb_ref, w2_ref,
                 b2_ref, out_ref):
    cr = cnt_ref[...]
    ar = agg_ref[...]
    cnt = cr[0, :, 0:1] + cr[1, :, 0:1]
    inv = 1.0 / jnp.maximum(cnt, 1.0)
    agg = (ar[0] + ar[1]) * inv
    hr = lax.dot_general(h_ref[...], root_ref[...],
                         (((1,), (0,)), ((), ())), preferred_element_type=F32)
    h = jax.nn.relu(hr + agg + cb_ref[...])
    if last:
        out_ref[...] = lax.dot_general(
            h, w2_ref[...], (((1,), (0,)), ((), ())),
            preferred_element_type=F32) + b2_ref[...]
    else:
        out_ref[...] = h


def _update_call(n_pad, last, h, aggp, cntp, root, conv_b, fc2_W, fc2_b):
    grid = n_pad // TN
    width_out = 1 if last else WIDTH
    return pl.pallas_call(
        functools.partial(_update_body, last),
        grid=(grid,),
        in_specs=[
            pl.BlockSpec((TN, WIDTH), lambda i: (i, 0)),
            pl.BlockSpec((NC, TN, WIDTH), lambda i: (0, i, 0)),
            pl.BlockSpec((NC, TN, 16), lambda i: (0, i, 0)),
            pl.BlockSpec((WIDTH, WIDTH), lambda i: (0, 0)),
            pl.BlockSpec((1, WIDTH), lambda i: (0, 0)),
            pl.BlockSpec((WIDTH, 1), lambda i: (0, 0)),
            pl.BlockSpec((1, 1), lambda i: (0, 0)),
        ],
        out_specs=pl.BlockSpec((TN, width_out), lambda i: (i, 0)),
        out_shape=jax.ShapeDtypeStruct((n_pad, width_out), F32),
    )(h, aggp, cntp, root, conv_b, fc2_W, fc2_b)


def _fc1_body(x_ref, w_ref, b_ref, out_ref):
    out_ref[...] = x_ref[...] * w_ref[...] + b_ref[...]


def _kh_body(ea_ref, w_ref, b_ref, out_ref):
    out_ref[...] = jax.nn.relu(
        lax.dot_general(ea_ref[...], w_ref[...],
                        (((1,), (0,)), ((), ())), preferred_element_type=F32)
        + b_ref[...])


def kernel(x, edge_index, edge_attr, fc1_W, fc1_b, k_W1, k_b1, k_W2, k_b2,
           root, conv_b, fc2_W, fc2_b):
    n, _ = x.shape
    e = edge_index.shape[1]
    ker_in = edge_attr.shape[1]
    depth = 4

    n_pad = ((n + 1 + NS * CH - 1) // (NS * CH)) * (NS * CH)
    e_pad = ((e + NW * SB - 1) // (NW * SB)) * (NW * SB)

    src = edge_index[0].astype(jnp.int32)
    dst = edge_index[1].astype(jnp.int32)
    src_p = jnp.concatenate([src, jnp.zeros((e_pad - e,), jnp.int32)])
    dst_p = jnp.concatenate([dst, jnp.full((e_pad - e,), n, jnp.int32)])
    src3 = src_p.reshape(NW, e_pad // (NW * CH), CH)
    dst3 = dst_p.reshape(NW, e_pad // (NW * CH), CH)

    ea = jnp.concatenate(
        [edge_attr, jnp.ones((e, 1), F32), jnp.zeros((e, 3), F32)], axis=1)
    ea = jnp.concatenate([ea, jnp.zeros((e_pad - e, ker_in + 4), F32)])
    w1p = jnp.zeros((ker_in + 4, KR), F32)
    w1p = w1p.at[:ker_in, :WIDTH // 4].set(k_W1)
    w1p = w1p.at[ker_in, WIDTH // 4].set(1.0)
    b1p = jnp.zeros((1, KR), F32).at[0, :WIDTH // 4].set(k_b1)

    a_stack = jnp.concatenate([k_W2, k_b2[None, :]], axis=0)
    a_stack = a_stack.reshape(9 * WIDTH, WIDTH)

    x_p = jnp.concatenate([x, jnp.zeros((n_pad - n, 1), F32)])
    zeros32 = jnp.zeros((CH, WIDTH), F32)
    zeros16 = jnp.zeros((CH, 16), F32)
    ones16 = jnp.ones((CH, 16), F32)

    h = pl.pallas_call(
        _fc1_body,
        grid=(n_pad // TN,),
        in_specs=[
            pl.BlockSpec((TN, 1), lambda i: (i, 0)),
            pl.BlockSpec((1, WIDTH), lambda i: (0, 0)),
            pl.BlockSpec((1, WIDTH), lambda i: (0, 0)),
        ],
        out_specs=pl.BlockSpec((TN, WIDTH), lambda i: (i, 0)),
        out_shape=jax.ShapeDtypeStruct((n_pad, WIDTH), F32),
    )(x_p, fc1_W, fc1_b[None, :])

    kh_aug = pl.pallas_call(
        _kh_body,
        grid=(e_pad // TE,),
        in_specs=[
            pl.BlockSpec((TE, ker_in + 4), lambda i: (i, 0)),
            pl.BlockSpec((ker_in + 4, KR), lambda i: (0, 0)),
            pl.BlockSpec((1, KR), lambda i: (0, 0)),
        ],
        out_specs=pl.BlockSpec((TE, KR), lambda i: (i, 0)),
        out_shape=jax.ShapeDtypeStruct((e_pad, KR), F32),
    )(ea, w1p, b1p)

    gather = _make_gather(n_pad, e_pad)
    scatter = _make_scatter(n_pad, e_pad, WIDTH)
    count = _make_count(n_pad, e_pad)

    cntp = count(dst3, zeros16, ones16)

    conv_b2 = conv_b[None, :]
    fc2_b2 = fc2_b[None, :]
    for layer in range(depth):
        hs = gather(h, src3)
        msg = _msg_call(e_pad, hs, kh_aug, a_stack)
        aggp = scatter(msg, dst3, zeros32)
        h = _update_call(n_pad, layer == depth - 1, h, aggp, cntp,
                         root, conv_b2, fc2_W, fc2_b2)

    return h[:n]
